# baseline (speedup 1.0000x reference)
import functools

import jax
import jax.numpy as jnp
from jax import lax
from jax.experimental import pallas as pl
from jax.experimental.pallas import tpu as pltpu
from jax.experimental.pallas import tpu_sc as plsc

N_NODES = 10000
N_EDGES = 320000
D = 128

NC = 2
NS = 16
NW = NC * NS
E_PER_W = N_EDGES // NW
C = 80
NCH = E_PER_W // C
RPT = 640
RPT_LAST = N_NODES - (NS - 1) * RPT


def _sc_body(x_hbm, src_hbm, dst_hbm, attr_hbm, wedge_hbm, zeros_hbm, out_hbm,
             wedge_v, src_v, dst_v, attr_v, rows_v, agg_sh, sem):
    cid = lax.axis_index("c")
    sid = lax.axis_index("s")
    wid = cid * NS + sid
    base = wid * E_PER_W

    pltpu.sync_copy(wedge_hbm, wedge_v)

    @pl.when(sid < NS - 1)
    def _():
        pltpu.sync_copy(zeros_hbm.at[pl.ds(sid * RPT, RPT)],
                        agg_sh.at[pl.ds(sid * RPT, RPT)])

    @pl.when(sid == NS - 1)
    def _():
        pltpu.sync_copy(zeros_hbm.at[pl.ds((NS - 1) * RPT, RPT_LAST)],
                        agg_sh.at[pl.ds((NS - 1) * RPT, RPT_LAST)])

    plsc.subcore_barrier()

    def chunk_body(i, carry):
        off = base + i * C
        pltpu.sync_copy(src_hbm.at[pl.ds(off, C)], src_v)
        pltpu.sync_copy(dst_hbm.at[pl.ds(off, C)], dst_v)
        pltpu.sync_copy(attr_hbm.at[pl.ds(off * 4, C * 4)],
                        attr_v.at[pl.ds(0, C * 4)])
        pltpu.async_copy(x_hbm.at[src_v], rows_v, sem).wait()

        def edge_body(e, c2):
            av = attr_v[pl.ds(e * 4, 16)]
            a0 = jnp.full((16,), av[0], jnp.float32)
            a1 = jnp.full((16,), av[1], jnp.float32)
            a2 = jnp.full((16,), av[2], jnp.float32)
            a3 = jnp.full((16,), av[3], jnp.float32)
            for j in range(D // 16):
                sl = pl.ds(j * 16, 16)
                z = (a0 * wedge_v[0, sl] + a1 * wedge_v[1, sl]
                     + a2 * wedge_v[2, sl] + a3 * wedge_v[3, sl])
                rows_v[e, sl] = rows_v[e, sl] / (1.0 + jnp.exp(-z))
            return c2

        lax.fori_loop(0, C, edge_body, 0)
        pltpu.sync_copy(rows_v, agg_sh.at[dst_v], add=True)
        return carry

    lax.fori_loop(0, NCH, chunk_body, 0)
    plsc.subcore_barrier()

    @pl.when(sid < NS - 1)
    def _():
        pltpu.sync_copy(agg_sh.at[pl.ds(sid * RPT, RPT)],
                        out_hbm.at[pl.ds(cid * N_NODES + sid * RPT, RPT)])

    @pl.when(sid == NS - 1)
    def _():
        pltpu.sync_copy(agg_sh.at[pl.ds((NS - 1) * RPT, RPT_LAST)],
                        out_hbm.at[pl.ds(cid * N_NODES + (NS - 1) * RPT, RPT_LAST)])


@jax.jit
def _sc_aggregate(x, src, dst, edge_attr, W_edge, zeros):
    mesh = plsc.VectorSubcoreMesh(core_axis_name="c", subcore_axis_name="s")
    return pl.kernel(
        _sc_body,
        mesh=mesh,
        out_type=jax.ShapeDtypeStruct((NC * N_NODES, D), jnp.float32),
        scratch_types=[
            pltpu.VMEM((4, D), jnp.float32),
            pltpu.VMEM((C,), jnp.int32),
            pltpu.VMEM((C,), jnp.int32),
            pltpu.VMEM((C * 4 + 16,), jnp.float32),
            pltpu.VMEM((C, D), jnp.float32),
            pltpu.VMEM_SHARED((N_NODES, D), jnp.float32),
            pltpu.SemaphoreType.DMA,
        ],
    )(x, src, dst, edge_attr, W_edge, zeros)


def _tc_body(a0_ref, a1_ref, w_ref, b_ref, out_ref):
    agg = a0_ref[...] + a1_ref[...]
    y = jnp.dot(agg, w_ref[...], preferred_element_type=jnp.float32)
    out_ref[...] = jnp.tanh(y + b_ref[...])


@jax.jit
def _tc_update(agg2, W, b2):
    B = 1000
    nb = N_NODES // B
    return pl.pallas_call(
        _tc_body,
        grid=(nb,),
        in_specs=[
            pl.BlockSpec((B, D), lambda i: (i, 0)),
            pl.BlockSpec((B, D), lambda i: (i + nb, 0)),
            pl.BlockSpec((D, D), lambda i: (0, 0)),
            pl.BlockSpec((1, D), lambda i: (0, 0)),
        ],
        out_specs=pl.BlockSpec((B, D), lambda i: (i, 0)),
        out_shape=jax.ShapeDtypeStruct((N_NODES, D), jnp.float32),
    )(agg2, agg2, W, b2)


def kernel(x, edge_index, edge_attr, W_edge, W, b):
    src = edge_index[0].astype(jnp.int32)
    dst = edge_index[1].astype(jnp.int32)
    zeros = jnp.zeros((N_NODES, D), jnp.float32)
    agg2 = _sc_aggregate(x, src, dst, edge_attr.reshape(-1), W_edge, zeros)
    return _tc_update(agg2, W, b.reshape(1, D))


# speedup vs baseline: 3.6026x; 3.6026x over previous
import functools

import jax
import jax.numpy as jnp
from jax import lax
from jax.experimental import pallas as pl
from jax.experimental.pallas import tpu as pltpu
from jax.experimental.pallas import tpu_sc as plsc

N_NODES = 10000
N_EDGES = 320000
D = 128

NC = 2
NS = 16
NW = NC * NS
E_PER_W = N_EDGES // NW
C = 80
NCH = E_PER_W // C
RPT = 640
RPT_LAST = N_NODES - (NS - 1) * RPT


def _sc_body(x_hbm, src_hbm, dst_hbm, attr_hbm, wedge_hbm, zeros_hbm, out_hbm,
             wedge_v, src_all, dst_all, attr_a, attr_b, rows_v, agg_sh,
             sem0, sem1, asem0, asem1):
    cid = lax.axis_index("c")
    sid = lax.axis_index("s")
    wid = cid * NS + sid

    pltpu.sync_copy(wedge_hbm, wedge_v)
    pltpu.sync_copy(src_hbm.at[pl.ds(wid * E_PER_W, E_PER_W)], src_all)
    pltpu.sync_copy(dst_hbm.at[wid], dst_all)

    @pl.when(sid < NS - 1)
    def _():
        pltpu.sync_copy(zeros_hbm.at[pl.ds(sid * RPT, RPT)],
                        agg_sh.at[pl.ds(sid * RPT, RPT)])

    @pl.when(sid == NS - 1)
    def _():
        pltpu.sync_copy(zeros_hbm.at[pl.ds((NS - 1) * RPT, RPT_LAST)],
                        agg_sh.at[pl.ds((NS - 1) * RPT, RPT_LAST)])

    plsc.subcore_barrier()

    wn = [[-wedge_v[k, pl.ds(j * 16, 16)] for k in range(4)]
          for j in range(D // 16)]
    sems = (sem0, sem1)
    asems = (asem0, asem1)
    attrs = (attr_a, attr_b)

    def gather(i, b):
        pltpu.async_copy(x_hbm.at[src_all.at[pl.ds(i * C, C)]],
                         rows_v.at[b], sems[b])
        pltpu.async_copy(
            attr_hbm.at[pl.ds((wid * E_PER_W + i * C) * 4, C * 4)],
            attrs[b].at[pl.ds(0, C * 4)], asems[b])

    def compute_scatter(i, b):
        pltpu.make_async_copy(x_hbm.at[src_all.at[pl.ds(0, C)]],
                              rows_v.at[b], sems[b]).wait()
        pltpu.make_async_copy(attr_hbm.at[pl.ds(0, C * 4)],
                              attrs[b].at[pl.ds(0, C * 4)], asems[b]).wait()

        @plsc.parallel_loop(0, C, unroll=4)
        def _edges(e):
            av = attrs[b][pl.ds(e * 4, 16)]
            a = [jnp.full((16,), av[kk], jnp.float32) for kk in range(4)]
            for j in range(D // 16):
                sl = pl.ds(j * 16, 16)
                z = (a[0] * wn[j][0] + a[1] * wn[j][1]
                     + a[2] * wn[j][2] + a[3] * wn[j][3])
                rows_v[b, e, sl] = rows_v[b, e, sl] / (1.0 + jnp.exp(z))

        pltpu.sync_copy(rows_v.at[b], agg_sh.at[dst_all.at[i]], add=True)

    gather(0, 0)

    def pair_body(p, carry):
        i0 = p * 2
        gather(i0 + 1, 1)
        compute_scatter(i0, 0)
        gather(i0 + 2, 0)
        compute_scatter(i0 + 1, 1)
        return carry

    lax.fori_loop(0, (NCH - 1) // 2, pair_body, 0)
    compute_scatter(NCH - 1, 0)
    plsc.subcore_barrier()

    @pl.when(sid < NS - 1)
    def _():
        pltpu.sync_copy(agg_sh.at[pl.ds(sid * RPT, RPT)],
                        out_hbm.at[pl.ds(cid * N_NODES + sid * RPT, RPT)])

    @pl.when(sid == NS - 1)
    def _():
        pltpu.sync_copy(agg_sh.at[pl.ds((NS - 1) * RPT, RPT_LAST)],
                        out_hbm.at[pl.ds(cid * N_NODES + (NS - 1) * RPT, RPT_LAST)])


@jax.jit
def _sc_aggregate(x, src3, dst3, attr3, W_edge, zeros):
    mesh = plsc.VectorSubcoreMesh(core_axis_name="c", subcore_axis_name="s")
    return pl.kernel(
        _sc_body,
        mesh=mesh,
        out_type=jax.ShapeDtypeStruct((NC * N_NODES, D), jnp.float32),
        scratch_types=[
            pltpu.VMEM((4, D), jnp.float32),
            pltpu.VMEM((E_PER_W,), jnp.int32),
            pltpu.VMEM((NCH, C), jnp.int32),
            pltpu.VMEM((C * 4 + 16,), jnp.float32),
            pltpu.VMEM((C * 4 + 16,), jnp.float32),
            pltpu.VMEM((2, C, D), jnp.float32),
            pltpu.VMEM_SHARED((N_NODES, D), jnp.float32),
            pltpu.SemaphoreType.DMA,
            pltpu.SemaphoreType.DMA,
            pltpu.SemaphoreType.DMA,
            pltpu.SemaphoreType.DMA,
        ],
    )(x, src3, dst3, attr3, W_edge, zeros)


def _tc_body(a0_ref, a1_ref, w_ref, b_ref, out_ref):
    agg = a0_ref[...] + a1_ref[...]
    y = jnp.dot(agg, w_ref[...], preferred_element_type=jnp.float32)
    out_ref[...] = jnp.tanh(y + b_ref[...])


@jax.jit
def _tc_update(agg2, W, b2):
    B = 1000
    nb = N_NODES // B
    return pl.pallas_call(
        _tc_body,
        grid=(nb,),
        in_specs=[
            pl.BlockSpec((B, D), lambda i: (i, 0)),
            pl.BlockSpec((B, D), lambda i: (i + nb, 0)),
            pl.BlockSpec((D, D), lambda i: (0, 0)),
            pl.BlockSpec((1, D), lambda i: (0, 0)),
        ],
        out_specs=pl.BlockSpec((B, D), lambda i: (i, 0)),
        out_shape=jax.ShapeDtypeStruct((N_NODES, D), jnp.float32),
    )(agg2, agg2, W, b2)


def kernel(x, edge_index, edge_attr, W_edge, W, b):
    src3 = edge_index[0].astype(jnp.int32)
    dst3 = edge_index[1].astype(jnp.int32).reshape(NW, NCH, C)
    attr3 = edge_attr.reshape(-1)
    zeros = jnp.zeros((N_NODES, D), jnp.float32)
    agg2 = _sc_aggregate(x, src3, dst3, attr3, W_edge, zeros)
    return _tc_update(agg2, W, b.reshape(1, D))
